# baseline (speedup 1.0000x reference)
import functools

import jax
import jax.numpy as jnp
from jax import lax
from jax.experimental import pallas as pl
from jax.experimental.pallas import tpu as pltpu
from jax.experimental.pallas import tpu_sc as plsc

N = 10000
E = 320000
D_IN = 128
D_H = 16
D_OUT = 3

NC = 2
NS = 16
NW = NC * NS
B = 128
R = E // B
CH = 10
NCHUNK = R // CH
C0 = 134
C1 = NCHUNK - C0
MAXQ0 = 9
MAXQ1 = 8
NB = 4
RD0 = 1900
RD1 = R - RD0
MAXR0 = 119
MAXR1 = 38
NPAD = 10240
ZCH = NPAD // NS // B
TSA = N // NS
ZRA = TSA // 5
BLK = 1000

_mesh = plsc.VectorSubcoreMesh(core_axis_name="c", subcore_axis_name="s")
_sc_params = pltpu.CompilerParams(
    use_tc_tiling_on_sc=False, disable_bounds_checks=True)


@functools.partial(
    pl.kernel,
    out_type=jax.ShapeDtypeStruct((NC, NPAD), jnp.float32),
    mesh=_mesh,
    scratch_types=[
        pltpu.VMEM((MAXR0, B), jnp.int32),
        pltpu.VMEM((B,), jnp.float32),
        pltpu.VMEM_SHARED((NPAD,), jnp.float32),
        pltpu.SemaphoreType.DMA,
    ],
    compiler_params=_sc_params,
)
def _deg_kernel(dst_hbm, out_hbm, dstv, val, degsh, sem):
    c = lax.axis_index("c")
    s = lax.axis_index("s")
    for l in range(B // 16):
        val[pl.ds(l * 16, 16)] = jnp.zeros((16,), jnp.float32)
    for i in range(ZCH):
        pltpu.sync_copy(val, degsh.at[pl.ds(s * (NPAD // NS) + i * B, B)])
    plsc.subcore_barrier()

    def run(lo, nrows, maxr):
        pltpu.sync_copy(dst_hbm.at[pl.ds(lo, maxr)], dstv.at[pl.ds(0, maxr)])
        for l in range(B // 16):
            val[pl.ds(l * 16, 16)] = jnp.ones((16,), jnp.float32)

        def body(j, carry):
            pltpu.async_copy(val, degsh.at[dstv.at[j]], sem, add=True)
            return carry

        lax.fori_loop(0, nrows, body, 0)

        def drain(j, carry):
            pltpu.make_async_copy(val, degsh.at[dstv.at[j]], sem).wait()
            return carry

        lax.fori_loop(0, nrows, drain, 0)

    @pl.when(c == 0)
    def _():
        lo = (s * RD0) // NS
        hi = ((s + 1) * RD0) // NS
        run(lo, hi - lo, MAXR0)

    @pl.when(c == 1)
    def _():
        lo = RD0 + (s * RD1) // NS
        hi = RD0 + ((s + 1) * RD1) // NS
        run(lo, hi - lo, MAXR1)

    plsc.subcore_barrier()
    ts = NPAD // NS
    pltpu.sync_copy(degsh.at[pl.ds(s * ts, ts)], out_hbm.at[c].at[pl.ds(s * ts, ts)])


@functools.partial(
    pl.kernel,
    out_type=jax.ShapeDtypeStruct((NC, N, D_H), jnp.float32),
    mesh=_mesh,
    scratch_types=[
        pltpu.VMEM((MAXQ0 * CH * B,), jnp.int32),
        pltpu.VMEM((MAXQ0 * CH, B), jnp.int32),
        [pltpu.VMEM((CH * B, D_H), jnp.float32) for _ in range(NB)],
        pltpu.VMEM_SHARED((N, D_H), jnp.float32),
        [pltpu.SemaphoreType.DMA for _ in range(NB)],
        [pltpu.SemaphoreType.DMA for _ in range(NB)],
    ],
    compiler_params=_sc_params,
)
def _agg_kernel(y_hbm, src_hbm, dst_hbm, out_hbm, srcv, dstv, rows, aggsh, gsem, ssem):
    c = lax.axis_index("c")
    s = lax.axis_index("s")

    def zrow(i, carry):
        rows[0][i] = jnp.zeros((16,), jnp.float32)
        return carry

    lax.fori_loop(0, B, zrow, 0)
    for i in range(5):
        pltpu.sync_copy(rows[0].at[pl.ds(0, ZRA)], aggsh.at[pl.ds(s * TSA + i * ZRA, ZRA)])
    plsc.subcore_barrier()

    def run(lo, nch, maxq):
        pltpu.sync_copy(src_hbm.at[pl.ds(lo * CH * B, maxq * CH * B)],
                        srcv.at[pl.ds(0, maxq * CH * B)])
        pltpu.sync_copy(dst_hbm.at[pl.ds(lo * CH, maxq * CH)],
                        dstv.at[pl.ds(0, maxq * CH)])

        def gissue(q, b):
            pltpu.async_copy(
                y_hbm.at[srcv.at[pl.ds(q * CH * B, CH * B)]], rows[b], gsem[b])

        def gwait(q, b):
            pltpu.make_async_copy(
                y_hbm.at[srcv.at[pl.ds(q * CH * B, CH * B)]], rows[b], gsem[b]).wait()

        def sissue(q, b):
            def f(j, carry):
                pltpu.async_copy(rows[b].at[pl.ds((j - q * CH) * B, B)],
                                 aggsh.at[dstv.at[j]], ssem[b], add=True)
                return carry
            lax.fori_loop(q * CH, (q + 1) * CH, f, 0)

        def sdrain(q, b):
            def f(j, carry):
                pltpu.make_async_copy(rows[b].at[pl.ds((j - q * CH) * B, B)],
                                      aggsh.at[dstv.at[j]], ssem[b]).wait()
                return carry
            lax.fori_loop(q * CH, (q + 1) * CH, f, 0)

        for b in range(NB):
            @pl.when(b < nch)
            def _(b=b):
                gissue(b, b)
        for q in range(maxq):
            @pl.when(q < nch)
            def _(q=q, b=q % NB):
                gwait(q, b)
                sissue(q, b)

            if q + NB < maxq:
                @pl.when(q + NB < nch)
                def _(q=q, b=q % NB):
                    sdrain(q, b)
                    gissue(q + NB, b)
        for q in range(maxq):
            @pl.when((q >= nch - NB) & (q < nch))
            def _(q=q, b=q % NB):
                sdrain(q, b)

    @pl.when(c == 0)
    def _():
        lo = (s * C0) // NS
        hi = ((s + 1) * C0) // NS
        run(lo, hi - lo, MAXQ0)

    @pl.when(c == 1)
    def _():
        lo = C0 + (s * C1) // NS
        hi = C0 + ((s + 1) * C1) // NS
        run(lo, hi - lo, MAXQ1)

    plsc.subcore_barrier()

    pltpu.sync_copy(aggsh.at[pl.ds(s * TSA, TSA)], out_hbm.at[c].at[pl.ds(s * TSA, TSA)])


def _mm1(x_ref, w1_ref, xw_ref):
    xw_ref[...] = jnp.dot(x_ref[...], w1_ref[...], preferred_element_type=jnp.float32)


def _stage1(xw_ref, d0_ref, d1_ref, y_ref, dis_ref):
    deg = d0_ref[...] + d1_ref[...] + 1.0
    dis = lax.rsqrt(deg)
    y_ref[...] = xw_ref[...] * dis
    dis_ref[...] = dis


def _stage2(agg_ref, y_ref, dis_ref, b1_ref, w2_ref, y2_ref):
    dis = dis_ref[...]
    h = (agg_ref[0] + agg_ref[1] + y_ref[...]) * dis + b1_ref[...]
    h = jnp.maximum(h, 0.0)
    y2_ref[...] = jnp.dot(h, w2_ref[...], preferred_element_type=jnp.float32) * dis


def _stage3(agg_ref, y2_ref, dis_ref, b2_ref, out_ref):
    z = (agg_ref[0] + agg_ref[1] + y2_ref[...]) * dis_ref[...]
    z3 = z[:, :D_OUT] + b2_ref[...]
    m = jnp.max(z3, axis=1, keepdims=True)
    e = jnp.exp(z3 - m)
    ssum = jnp.sum(e, axis=1, keepdims=True)
    out_ref[...] = z3 - m - jnp.log(ssum)


_row_spec = pl.BlockSpec((BLK, D_H), lambda i: (i, 0))
_dis_spec = pl.BlockSpec((BLK, 1), lambda i: (i, 0))
_agg_spec = pl.BlockSpec((NC, BLK, D_H), lambda i: (0, i, 0))

_mm1_call = pl.pallas_call(
    _mm1,
    grid=(N // BLK,),
    in_specs=[
        pl.BlockSpec((BLK, D_IN), lambda i: (i, 0)),
        pl.BlockSpec((D_IN, D_H), lambda i: (0, 0)),
    ],
    out_specs=_row_spec,
    out_shape=jax.ShapeDtypeStruct((N, D_H), jnp.float32),
)

_stage1_call = pl.pallas_call(
    _stage1,
    grid=(N // BLK,),
    in_specs=[_row_spec, _dis_spec, _dis_spec],
    out_specs=(_row_spec, _dis_spec),
    out_shape=(
        jax.ShapeDtypeStruct((N, D_H), jnp.float32),
        jax.ShapeDtypeStruct((N, 1), jnp.float32),
    ),
)

_stage2_call = pl.pallas_call(
    _stage2,
    grid=(N // BLK,),
    in_specs=[
        _agg_spec,
        _row_spec,
        _dis_spec,
        pl.BlockSpec((1, D_H), lambda i: (0, 0)),
        pl.BlockSpec((D_H, D_H), lambda i: (0, 0)),
    ],
    out_specs=_row_spec,
    out_shape=jax.ShapeDtypeStruct((N, D_H), jnp.float32),
)

_stage3_call = pl.pallas_call(
    _stage3,
    grid=(N // BLK,),
    in_specs=[
        _agg_spec,
        _row_spec,
        _dis_spec,
        pl.BlockSpec((1, D_OUT), lambda i: (0, 0)),
    ],
    out_specs=pl.BlockSpec((BLK, D_OUT), lambda i: (i, 0)),
    out_shape=jax.ShapeDtypeStruct((N, D_OUT), jnp.float32),
)


def kernel(x, edge_index, W1, b1, W2, b2):
    ei = edge_index.astype(jnp.int32)
    src_f = ei[0]
    dst_r = ei[1].reshape(R, B)

    xw = _mm1_call(x, W1)
    deg2 = _deg_kernel(dst_r)
    d0 = deg2[0, :N].reshape(N, 1)
    d1 = deg2[1, :N].reshape(N, 1)

    y, dis = _stage1_call(xw, d0, d1)

    agg = _agg_kernel(y, src_f, dst_r)

    w2p = jnp.concatenate([W2, jnp.zeros((D_H, D_H - D_OUT), jnp.float32)], axis=1)
    y2 = _stage2_call(agg, y, dis, b1.reshape(1, D_H), w2p)

    agg2 = _agg_kernel(y2, src_f, dst_r)
    out = _stage3_call(agg2, y2, dis, b2.reshape(1, D_OUT))
    return out


# speedup vs baseline: 1.0506x; 1.0506x over previous
import functools

import jax
import jax.numpy as jnp
from jax import lax
from jax.experimental import pallas as pl
from jax.experimental.pallas import tpu as pltpu
from jax.experimental.pallas import tpu_sc as plsc

N = 10000
E = 320000
D_IN = 128
D_H = 16
D_OUT = 3

NC = 2
NS = 16
NW = NC * NS
B = 128
R = E // B
CH = 10
NCHUNK = R // CH
C0 = 134
C1 = NCHUNK - C0
MAXQ0 = 9
MAXQ1 = 8
NB = 4
RD0 = 1900
RD1 = R - RD0
MAXR0 = 119
MAXR1 = 38
NPAD = 10240
ZCH = NPAD // NS // B
TSA = N // NS
ZRA = TSA // 5
BLK = 2000

_mesh = plsc.VectorSubcoreMesh(core_axis_name="c", subcore_axis_name="s")
_sc_params = pltpu.CompilerParams(
    use_tc_tiling_on_sc=False, disable_bounds_checks=True)


@functools.partial(
    pl.kernel,
    out_type=jax.ShapeDtypeStruct((NC, NPAD), jnp.float32),
    mesh=_mesh,
    scratch_types=[
        pltpu.VMEM((MAXR0, B), jnp.int32),
        pltpu.VMEM((B,), jnp.float32),
        pltpu.VMEM_SHARED((NPAD,), jnp.float32),
        pltpu.SemaphoreType.DMA,
    ],
    compiler_params=_sc_params,
)
def _deg_kernel(dst_hbm, out_hbm, dstv, val, degsh, sem):
    c = lax.axis_index("c")
    s = lax.axis_index("s")
    for l in range(B // 16):
        val[pl.ds(l * 16, 16)] = jnp.zeros((16,), jnp.float32)
    for i in range(ZCH):
        pltpu.sync_copy(val, degsh.at[pl.ds(s * (NPAD // NS) + i * B, B)])
    plsc.subcore_barrier()

    def run(lo, nrows, maxr):
        pltpu.sync_copy(dst_hbm.at[pl.ds(lo, maxr)], dstv.at[pl.ds(0, maxr)])
        for l in range(B // 16):
            val[pl.ds(l * 16, 16)] = jnp.ones((16,), jnp.float32)

        def body(j, carry):
            pltpu.async_copy(val, degsh.at[dstv.at[j]], sem, add=True)
            return carry

        lax.fori_loop(0, nrows, body, 0)

        def drain(j, carry):
            pltpu.make_async_copy(val, degsh.at[dstv.at[j]], sem).wait()
            return carry

        lax.fori_loop(0, nrows, drain, 0)

    @pl.when(c == 0)
    def _():
        lo = (s * RD0) // NS
        hi = ((s + 1) * RD0) // NS
        run(lo, hi - lo, MAXR0)

    @pl.when(c == 1)
    def _():
        lo = RD0 + (s * RD1) // NS
        hi = RD0 + ((s + 1) * RD1) // NS
        run(lo, hi - lo, MAXR1)

    plsc.subcore_barrier()
    ts = NPAD // NS
    pltpu.sync_copy(degsh.at[pl.ds(s * ts, ts)], out_hbm.at[c].at[pl.ds(s * ts, ts)])


@functools.partial(
    pl.kernel,
    out_type=jax.ShapeDtypeStruct((NC, N, D_H), jnp.float32),
    mesh=_mesh,
    scratch_types=[
        pltpu.VMEM((MAXQ0 * CH * B,), jnp.int32),
        pltpu.VMEM((MAXQ0 * CH, B), jnp.int32),
        [pltpu.VMEM((CH * B, D_H), jnp.float32) for _ in range(NB)],
        pltpu.VMEM_SHARED((N, D_H), jnp.float32),
        [pltpu.SemaphoreType.DMA for _ in range(NB)],
        [pltpu.SemaphoreType.DMA for _ in range(NB)],
    ],
    compiler_params=_sc_params,
)
def _agg_kernel(y_hbm, src_hbm, dst_hbm, out_hbm, srcv, dstv, rows, aggsh, gsem, ssem):
    c = lax.axis_index("c")
    s = lax.axis_index("s")

    def zrow(i, carry):
        rows[0][i] = jnp.zeros((16,), jnp.float32)
        return carry

    lax.fori_loop(0, B, zrow, 0)
    for i in range(5):
        pltpu.sync_copy(rows[0].at[pl.ds(0, ZRA)], aggsh.at[pl.ds(s * TSA + i * ZRA, ZRA)])
    plsc.subcore_barrier()

    def run(lo, nch, maxq):
        pltpu.sync_copy(src_hbm.at[pl.ds(lo * CH * B, maxq * CH * B)],
                        srcv.at[pl.ds(0, maxq * CH * B)])
        pltpu.sync_copy(dst_hbm.at[pl.ds(lo * CH, maxq * CH)],
                        dstv.at[pl.ds(0, maxq * CH)])

        def gissue(q, b):
            pltpu.async_copy(
                y_hbm.at[srcv.at[pl.ds(q * CH * B, CH * B)]], rows[b], gsem[b])

        def gwait(q, b):
            pltpu.make_async_copy(
                y_hbm.at[srcv.at[pl.ds(q * CH * B, CH * B)]], rows[b], gsem[b]).wait()

        def sissue(q, b):
            def f(j, carry):
                pltpu.async_copy(rows[b].at[pl.ds((j - q * CH) * B, B)],
                                 aggsh.at[dstv.at[j]], ssem[b], add=True)
                return carry
            lax.fori_loop(q * CH, (q + 1) * CH, f, 0)

        def sdrain(q, b):
            def f(j, carry):
                pltpu.make_async_copy(rows[b].at[pl.ds((j - q * CH) * B, B)],
                                      aggsh.at[dstv.at[j]], ssem[b]).wait()
                return carry
            lax.fori_loop(q * CH, (q + 1) * CH, f, 0)

        for b in range(NB):
            @pl.when(b < nch)
            def _(b=b):
                gissue(b, b)
        for q in range(maxq):
            @pl.when(q < nch)
            def _(q=q, b=q % NB):
                gwait(q, b)
                sissue(q, b)

            if q + NB < maxq:
                @pl.when(q + NB < nch)
                def _(q=q, b=q % NB):
                    sdrain(q, b)
                    gissue(q + NB, b)
        for q in range(maxq):
            @pl.when((q >= nch - NB) & (q < nch))
            def _(q=q, b=q % NB):
                sdrain(q, b)

    @pl.when(c == 0)
    def _():
        lo = (s * C0) // NS
        hi = ((s + 1) * C0) // NS
        run(lo, hi - lo, MAXQ0)

    @pl.when(c == 1)
    def _():
        lo = C0 + (s * C1) // NS
        hi = C0 + ((s + 1) * C1) // NS
        run(lo, hi - lo, MAXQ1)

    plsc.subcore_barrier()

    pltpu.sync_copy(aggsh.at[pl.ds(s * TSA, TSA)], out_hbm.at[c].at[pl.ds(s * TSA, TSA)])


def _mm1(x_ref, w1_ref, xw_ref):
    xw_ref[...] = jnp.dot(x_ref[...], w1_ref[...], preferred_element_type=jnp.float32)


def _stage1(xw_ref, d0_ref, d1_ref, y_ref, dis_ref):
    deg = d0_ref[...] + d1_ref[...] + 1.0
    dis = lax.rsqrt(deg)
    y_ref[...] = xw_ref[...] * dis
    dis_ref[...] = dis


def _stage2(agg_ref, y_ref, dis_ref, b1_ref, w2_ref, y2_ref):
    dis = dis_ref[...]
    h = (agg_ref[0] + agg_ref[1] + y_ref[...]) * dis + b1_ref[...]
    h = jnp.maximum(h, 0.0)
    y2_ref[...] = jnp.dot(h, w2_ref[...], preferred_element_type=jnp.float32) * dis


def _stage3(agg_ref, y2_ref, dis_ref, b2_ref, out_ref):
    z = (agg_ref[0] + agg_ref[1] + y2_ref[...]) * dis_ref[...]
    z3 = z[:, :D_OUT] + b2_ref[...]
    m = jnp.max(z3, axis=1, keepdims=True)
    e = jnp.exp(z3 - m)
    ssum = jnp.sum(e, axis=1, keepdims=True)
    out_ref[...] = z3 - m - jnp.log(ssum)


_row_spec = pl.BlockSpec((BLK, D_H), lambda i: (i, 0))
_dis_spec = pl.BlockSpec((BLK, 1), lambda i: (i, 0))
_agg_spec = pl.BlockSpec((NC, BLK, D_H), lambda i: (0, i, 0))

_mm1_call = pl.pallas_call(
    _mm1,
    grid=(N // BLK,),
    in_specs=[
        pl.BlockSpec((BLK, D_IN), lambda i: (i, 0)),
        pl.BlockSpec((D_IN, D_H), lambda i: (0, 0)),
    ],
    out_specs=_row_spec,
    out_shape=jax.ShapeDtypeStruct((N, D_H), jnp.float32),
)

_stage1_call = pl.pallas_call(
    _stage1,
    grid=(N // BLK,),
    in_specs=[_row_spec, _dis_spec, _dis_spec],
    out_specs=(_row_spec, _dis_spec),
    out_shape=(
        jax.ShapeDtypeStruct((N, D_H), jnp.float32),
        jax.ShapeDtypeStruct((N, 1), jnp.float32),
    ),
)

_stage2_call = pl.pallas_call(
    _stage2,
    grid=(N // BLK,),
    in_specs=[
        _agg_spec,
        _row_spec,
        _dis_spec,
        pl.BlockSpec((1, D_H), lambda i: (0, 0)),
        pl.BlockSpec((D_H, D_H), lambda i: (0, 0)),
    ],
    out_specs=_row_spec,
    out_shape=jax.ShapeDtypeStruct((N, D_H), jnp.float32),
)

_stage3_call = pl.pallas_call(
    _stage3,
    grid=(N // BLK,),
    in_specs=[
        _agg_spec,
        _row_spec,
        _dis_spec,
        pl.BlockSpec((1, D_OUT), lambda i: (0, 0)),
    ],
    out_specs=pl.BlockSpec((BLK, D_OUT), lambda i: (i, 0)),
    out_shape=jax.ShapeDtypeStruct((N, D_OUT), jnp.float32),
)


def kernel(x, edge_index, W1, b1, W2, b2):
    ei = edge_index.astype(jnp.int32)
    src_f = ei[0]
    dst_r = ei[1].reshape(R, B)

    xw = _mm1_call(x, W1)
    deg2 = _deg_kernel(dst_r)
    d0 = deg2[0, :N].reshape(N, 1)
    d1 = deg2[1, :N].reshape(N, 1)

    y, dis = _stage1_call(xw, d0, d1)

    agg = _agg_kernel(y, src_f, dst_r)

    w2p = jnp.concatenate([W2, jnp.zeros((D_H, D_H - D_OUT), jnp.float32)], axis=1)
    y2 = _stage2_call(agg, y, dis, b1.reshape(1, D_H), w2p)

    agg2 = _agg_kernel(y2, src_f, dst_r)
    out = _stage3_call(agg2, y2, dis, b2.reshape(1, D_OUT))
    return out
